# Initial kernel scaffold; baseline (speedup 1.0000x reference)
import jax
import jax.numpy as jnp
from jax.experimental import pallas as pl


def kernel(input_ids, attention_mask, embed_table):
    raise NotImplementedError("write your pallas kernel here")


# speedup vs baseline: 1.6279x; 1.6279x over previous
import functools

import jax
import jax.numpy as jnp
from jax import lax
from jax.experimental import pallas as pl
from jax.experimental.pallas import tpu as pltpu
from jax.experimental.pallas import tpu_sc as plsc

_INFO = plsc.get_sparse_core_info()
_NC, _NS = _INFO.num_cores, _INFO.num_subcores
_NW = _NC * _NS


def _make_gather(V, D, B, chunk):
    assert B % _NW == 0
    b_per_w = B // _NW
    assert b_per_w % chunk == 0
    n_chunks = b_per_w // chunk
    mesh = plsc.VectorSubcoreMesh(core_axis_name="c", subcore_axis_name="s")

    @functools.partial(
        pl.kernel,
        mesh=mesh,
        out_type=jax.ShapeDtypeStruct((B, D), jnp.float32),
        scratch_types=[
            pltpu.VMEM((n_chunks, chunk), jnp.int32),
            pltpu.VMEM((chunk, D), jnp.float32),
            pltpu.SemaphoreType.DMA,
        ],
    )
    def gather_kernel(table_hbm, ids_hbm, out_hbm, idx_v, rows_v, sem):
        wid = lax.axis_index("s") * _NC + lax.axis_index("c")
        base = wid * b_per_w
        pltpu.sync_copy(ids_hbm.at[wid], idx_v)

        def body(g, carry):
            pltpu.async_copy(table_hbm.at[idx_v.at[g]], rows_v, sem).wait()
            pltpu.sync_copy(rows_v, out_hbm.at[pl.ds(base + g * chunk, chunk)])
            return carry

        lax.fori_loop(0, n_chunks, body, 0)

    return gather_kernel


def kernel(input_ids, attention_mask, embed_table):
    del attention_mask
    V, D = embed_table.shape
    B_, S = input_ids.shape
    B = B_ * S
    chunk = 64
    ids = input_ids.reshape(_NW, (B // _NW) // chunk, chunk)
    out = _make_gather(V, D, B, chunk)(embed_table, ids)
    return out.reshape(B_, S, D)


# speedup vs baseline: 1.6734x; 1.0280x over previous
import functools

import jax
import jax.numpy as jnp
from jax import lax
from jax.experimental import pallas as pl
from jax.experimental.pallas import tpu as pltpu
from jax.experimental.pallas import tpu_sc as plsc

_INFO = plsc.get_sparse_core_info()
_NC, _NS = _INFO.num_cores, _INFO.num_subcores
_NW = _NC * _NS


def _make_gather(V, D, B, chunk):
    assert B % _NW == 0
    b_per_w = B // _NW
    assert b_per_w % chunk == 0
    n_chunks = b_per_w // chunk
    assert n_chunks % 2 == 0
    mesh = plsc.VectorSubcoreMesh(core_axis_name="c", subcore_axis_name="s")

    @functools.partial(
        pl.kernel,
        mesh=mesh,
        out_type=jax.ShapeDtypeStruct((B, D), jnp.float32),
        scratch_types=[
            pltpu.VMEM((n_chunks, chunk), jnp.int32),
            pltpu.VMEM((chunk, D), jnp.float32),
            pltpu.VMEM((chunk, D), jnp.float32),
            pltpu.SemaphoreType.DMA,
            pltpu.SemaphoreType.DMA,
            pltpu.SemaphoreType.DMA,
            pltpu.SemaphoreType.DMA,
        ],
    )
    def gather_kernel(table_hbm, ids_hbm, out_hbm, idx_v, buf0, buf1,
                      gsem0, gsem1, wsem0, wsem1):
        wid = lax.axis_index("s") * _NC + lax.axis_index("c")
        base = wid * b_per_w
        pltpu.sync_copy(ids_hbm.at[wid], idx_v)

        def gather(g, buf, gsem):
            pltpu.async_copy(table_hbm.at[idx_v.at[g]], buf, gsem)

        def wait_gather(g, buf, gsem):
            pltpu.make_async_copy(table_hbm.at[idx_v.at[g]], buf, gsem).wait()

        def write(g, buf, wsem):
            pltpu.async_copy(buf, out_hbm.at[pl.ds(base + g * chunk, chunk)],
                             wsem)

        def wait_write(g, buf, wsem):
            pltpu.make_async_copy(
                buf, out_hbm.at[pl.ds(base + g * chunk, chunk)], wsem).wait()

        gather(0, buf0, gsem0)
        gather(1, buf1, gsem1)

        def body(h, carry):
            g = h * 2
            wait_gather(g, buf0, gsem0)
            write(g, buf0, wsem0)
            wait_gather(g + 1, buf1, gsem1)
            write(g + 1, buf1, wsem1)

            @pl.when(g + 2 < n_chunks)
            def _refill():
                wait_write(g, buf0, wsem0)
                gather(g + 2, buf0, gsem0)
                wait_write(g + 1, buf1, wsem1)
                gather(g + 3, buf1, gsem1)

            return carry

        lax.fori_loop(0, n_chunks // 2, body, 0)
        wait_write(n_chunks - 2, buf0, wsem0)
        wait_write(n_chunks - 1, buf1, wsem1)

    return gather_kernel


def kernel(input_ids, attention_mask, embed_table):
    del attention_mask
    V, D = embed_table.shape
    B_, S = input_ids.shape
    B = B_ * S
    chunk = 32
    ids = input_ids.reshape(_NW, (B // _NW) // chunk, chunk)
    out = _make_gather(V, D, B, chunk)(embed_table, ids)
    return out.reshape(B_, S, D)
